# baseline (speedup 1.0000x reference)
import functools

import jax
import jax.numpy as jnp
from jax import lax
from jax.experimental import pallas as pl
from jax.experimental.pallas import tpu as pltpu
from jax.experimental.pallas import tpu_sc as plsc

N = 10000
H = 128
NLAYERS = 3
BR = 400
NBLK = N // BR

HALF = 5120
MP = 2 * HALF
RA = 10112
DUMMY = RA - 8
CHUNK = 32
NG = 16
GMAX = 40
CAPCH = 10240
CAPE = CAPCH * CHUNK


def _in_body(x_ref, w_ref, b_ref, o_ref):
  o_ref[...] = jnp.maximum(
      lax.dot_general(x_ref[...], w_ref[...], (((1,), (1,)), ((), ())),
                      preferred_element_type=jnp.float32) + b_ref[...], 0.0)


def _pre_body(h_ref, gw_ref, whh_ref, bhh_ref, m_ref, gh_ref):
  h = h_ref[...]
  m_ref[...] = lax.dot_general(h, gw_ref[...], (((1,), (0,)), ((), ())),
                               preferred_element_type=jnp.float32)
  gh_ref[...] = lax.dot_general(h, whh_ref[...], (((1,), (1,)), ((), ())),
                                preferred_element_type=jnp.float32) + bhh_ref[...]


def _post_body(a0_ref, a1_ref, h_ref, gh_ref, wih_ref, bih_ref, ho_ref):
  agg = a0_ref[...] + a1_ref[...]
  gi = lax.dot_general(agg, wih_ref[...], (((1,), (1,)), ((), ())),
                       preferred_element_type=jnp.float32) + bih_ref[...]
  gh = gh_ref[...]
  h = h_ref[...]
  r = jax.nn.sigmoid(gi[:, :H] + gh[:, :H])
  z = jax.nn.sigmoid(gi[:, H:2 * H] + gh[:, H:2 * H])
  n = jnp.tanh(gi[:, 2 * H:] + r * gh[:, 2 * H:])
  ho_ref[...] = (1.0 - z) * n + z * h


def _out_body(h_ref, w_ref, b_ref, o_ref):
  h = jnp.maximum(h_ref[...], 0.0)
  o_ref[...] = jnp.maximum(
      lax.dot_general(h, w_ref[...], (((1,), (1,)), ((), ())),
                      preferred_element_type=jnp.float32) + b_ref[...], 0.0)


def _row_spec(cols):
  return pl.BlockSpec((BR, cols), lambda i: (i, 0))


def _full_spec(shape):
  return pl.BlockSpec(shape, lambda i: tuple(0 for _ in shape))


def _make_sc_scatter():
  mesh = plsc.VectorSubcoreMesh(core_axis_name="c", subcore_axis_name="s")

  @functools.partial(
      pl.kernel,
      out_type=jax.ShapeDtypeStruct((2, RA, H), jnp.float32),
      mesh=mesh,
      scratch_types=[
          pltpu.VMEM((NG, CHUNK), jnp.int32),
          pltpu.VMEM((NG, CHUNK), jnp.int32),
          pltpu.VMEM((CHUNK, H), jnp.float32),
          pltpu.VMEM_SHARED((HALF, H), jnp.float32),
          pltpu.VMEM_SHARED((RA, H), jnp.float32),
      ],
  )
  def sc_scatter(m_hbm, src_hbm, dst_hbm, z_hbm, out_hbm,
                 src_v, dst_v, rows_v, m_sh, agg_sh):
    c = lax.axis_index("c")
    s = lax.axis_index("s")

    pltpu.sync_copy(m_hbm.at[pl.ds(c * HALF + s * 320, 320)],
                    m_sh.at[pl.ds(s * 320, 320)])
    for k in range(4):
      pltpu.sync_copy(z_hbm, agg_sh.at[pl.ds(s * 632 + k * 128, 128)])
    pltpu.sync_copy(z_hbm.at[pl.ds(0, 120)],
                    agg_sh.at[pl.ds(s * 632 + 512, 120)])
    plsc.subcore_barrier()

    def body(g, carry):
      r = (g * 16 + s) * NG
      pltpu.sync_copy(src_hbm.at[c, pl.ds(r, NG)], src_v)
      pltpu.sync_copy(dst_hbm.at[c, pl.ds(r, NG)], dst_v)
      head = dst_v[0, pl.ds(0, 16)]
      go = head[0] != DUMMY

      @pl.when(go)
      def _():
        for k in range(NG):
          pltpu.sync_copy(m_sh.at[src_v.at[k]], rows_v)
          pltpu.sync_copy(rows_v, agg_sh.at[dst_v.at[k]], add=True)

      return carry

    lax.fori_loop(0, GMAX, body, 0)
    plsc.subcore_barrier()

    for k in range(4):
      r0 = s * 632 + k * 128
      pltpu.sync_copy(agg_sh.at[pl.ds(r0, 128)], out_hbm.at[c, pl.ds(r0, 128)])
    r0 = s * 632 + 512
    pltpu.sync_copy(agg_sh.at[pl.ds(r0, 120)], out_hbm.at[c, pl.ds(r0, 120)])

  return sc_scatter


_make_sc_scatter = functools.cache(_make_sc_scatter)


def kernel(x, edge_index, W_in, b_in, ggnn_w, w_ih, w_hh, b_ih, b_hh,
           W_out, b_out):
  B = x.shape[0]
  xs = x.reshape(B * N, x.shape[2])
  src = edge_index[0]
  dst = edge_index[1]
  E = src.shape[0]

  mask = src < HALF
  rank0 = jnp.cumsum(mask) - 1
  rank1 = jnp.cumsum(~mask) - 1
  pos = jnp.where(mask, rank0, CAPE + rank1)
  src_loc = jnp.where(mask, src, src - HALF)
  srcP = jnp.zeros((2 * CAPE,), jnp.int32).at[pos].set(
      src_loc, unique_indices=True).reshape(2, CAPCH, CHUNK)
  dstP = jnp.full((2 * CAPE,), DUMMY, jnp.int32).at[pos].set(
      dst, unique_indices=True).reshape(2, CAPCH, CHUNK)
  z128 = jnp.zeros((128, H), jnp.float32)

  b_in2 = b_in.reshape(1, H)
  b_ih2 = b_ih.reshape(1, 3 * H)
  b_hh2 = b_hh.reshape(1, 3 * H)
  b_out2 = b_out.reshape(1, H)

  h = pl.pallas_call(
      _in_body,
      grid=(NBLK,),
      in_specs=[_row_spec(H), _full_spec((H, H)), _full_spec((1, H))],
      out_specs=_row_spec(H),
      out_shape=jax.ShapeDtypeStruct((N, H), jnp.float32),
  )(xs, W_in, b_in2)

  for i in range(NLAYERS):
    m, gh = pl.pallas_call(
        _pre_body,
        grid=(NBLK,),
        in_specs=[_row_spec(H), _full_spec((H, H)), _full_spec((3 * H, H)),
                  _full_spec((1, 3 * H))],
        out_specs=[_row_spec(H), _row_spec(3 * H)],
        out_shape=[jax.ShapeDtypeStruct((N, H), jnp.float32),
                   jax.ShapeDtypeStruct((N, 3 * H), jnp.float32)],
    )(h, ggnn_w[i], w_hh, b_hh2)

    m_p = jnp.concatenate([m, jnp.zeros((MP - N, H), jnp.float32)])
    aggs = _make_sc_scatter()(m_p, srcP, dstP, z128)

    h = pl.pallas_call(
        _post_body,
        grid=(NBLK,),
        in_specs=[_row_spec(H), _row_spec(H), _row_spec(H), _row_spec(3 * H),
                  _full_spec((3 * H, H)), _full_spec((1, 3 * H))],
        out_specs=_row_spec(H),
        out_shape=jax.ShapeDtypeStruct((N, H), jnp.float32),
    )(aggs[0], aggs[1], h, gh, w_ih, b_ih2)

  out = pl.pallas_call(
      _out_body,
      grid=(NBLK,),
      in_specs=[_row_spec(H), _full_spec((H, H)), _full_spec((1, H))],
      out_specs=_row_spec(H),
      out_shape=jax.ShapeDtypeStruct((N, H), jnp.float32),
  )(h, W_out, b_out2)

  return out.reshape(B, N, H)


# speedup vs baseline: 1.8509x; 1.8509x over previous
import functools

import jax
import jax.numpy as jnp
from jax import lax
from jax.experimental import pallas as pl
from jax.experimental.pallas import tpu as pltpu
from jax.experimental.pallas import tpu_sc as plsc

N = 10000
H = 128
NLAYERS = 3
BR = 400
NBLK = N // BR

NW = 32
CHUNK = 128
NCH = 80
EPT = CHUNK * NCH
EPAD = NW * EPT
R = 10240
RPT = R // 16


def _in_body(x_ref, w_ref, b_ref, o_ref):
  o_ref[...] = jnp.maximum(
      lax.dot_general(x_ref[...], w_ref[...], (((1,), (1,)), ((), ())),
                      preferred_element_type=jnp.float32) + b_ref[...], 0.0)


def _pre_body(h_ref, gw_ref, whh_ref, bhh_ref, m_ref, gh_ref):
  h = h_ref[...]
  m_ref[...] = lax.dot_general(h, gw_ref[...], (((1,), (0,)), ((), ())),
                               preferred_element_type=jnp.float32)
  gh_ref[...] = lax.dot_general(h, whh_ref[...], (((1,), (1,)), ((), ())),
                                preferred_element_type=jnp.float32) + bhh_ref[...]


def _post_body(a0_ref, a1_ref, h_ref, gh_ref, wih_ref, bih_ref, ho_ref):
  agg = a0_ref[...] + a1_ref[...]
  gi = lax.dot_general(agg, wih_ref[...], (((1,), (1,)), ((), ())),
                       preferred_element_type=jnp.float32) + bih_ref[...]
  gh = gh_ref[...]
  h = h_ref[...]
  r = jax.nn.sigmoid(gi[:, :H] + gh[:, :H])
  z = jax.nn.sigmoid(gi[:, H:2 * H] + gh[:, H:2 * H])
  n = jnp.tanh(gi[:, 2 * H:] + r * gh[:, 2 * H:])
  ho_ref[...] = (1.0 - z) * n + z * h


def _out_body(h_ref, w_ref, b_ref, o_ref):
  h = jnp.maximum(h_ref[...], 0.0)
  o_ref[...] = jnp.maximum(
      lax.dot_general(h, w_ref[...], (((1,), (1,)), ((), ())),
                      preferred_element_type=jnp.float32) + b_ref[...], 0.0)


def _row_spec(cols):
  return pl.BlockSpec((BR, cols), lambda i: (i, 0))


def _full_spec(shape):
  return pl.BlockSpec(shape, lambda i: tuple(0 for _ in shape))


def _make_sc_scatter():
  mesh = plsc.VectorSubcoreMesh(core_axis_name="c", subcore_axis_name="s")

  @functools.partial(
      pl.kernel,
      out_type=jax.ShapeDtypeStruct((2, R, H), jnp.float32),
      mesh=mesh,
      scratch_types=[
          pltpu.VMEM((NCH, CHUNK), jnp.int32),
          pltpu.VMEM((NCH, CHUNK), jnp.int32),
          pltpu.VMEM((CHUNK, H), jnp.float32),
          pltpu.VMEM_SHARED((R, H), jnp.float32),
          pltpu.SemaphoreType.DMA,
      ],
  )
  def sc_scatter(m_hbm, src_hbm, dst_hbm, z_hbm, out_hbm,
                 src_v, dst_v, rows_v, agg_sh, sem):
    c = lax.axis_index("c")
    s = lax.axis_index("s")
    wid = s * 2 + c

    for k in range(RPT // CHUNK):
      pltpu.sync_copy(z_hbm, agg_sh.at[pl.ds(s * RPT + k * CHUNK, CHUNK)])
    plsc.subcore_barrier()

    pltpu.sync_copy(src_hbm.at[pl.ds(wid * NCH, NCH)], src_v)
    pltpu.sync_copy(dst_hbm.at[pl.ds(wid * NCH, NCH)], dst_v)

    def body(j, carry):
      pltpu.async_copy(m_hbm.at[src_v.at[j]], rows_v, sem).wait()
      pltpu.sync_copy(rows_v, agg_sh.at[dst_v.at[j]], add=True)
      return carry

    lax.fori_loop(0, NCH, body, 0)
    plsc.subcore_barrier()

    for k in range(RPT // CHUNK):
      r0 = s * RPT + k * CHUNK
      pltpu.sync_copy(agg_sh.at[pl.ds(r0, CHUNK)],
                      out_hbm.at[c, pl.ds(r0, CHUNK)])

  return sc_scatter


_make_sc_scatter = functools.cache(_make_sc_scatter)


def kernel(x, edge_index, W_in, b_in, ggnn_w, w_ih, w_hh, b_ih, b_hh,
           W_out, b_out):
  B = x.shape[0]
  xs = x.reshape(B * N, x.shape[2])
  src = edge_index[0]
  dst = edge_index[1]
  E = src.shape[0]
  pad = EPAD - E
  src_p = jnp.concatenate([src, jnp.zeros((pad,), jnp.int32)]).reshape(-1, CHUNK)
  dst_p = jnp.concatenate([dst, jnp.full((pad,), R - 1, jnp.int32)]).reshape(-1, CHUNK)
  z128 = jnp.zeros((CHUNK, H), jnp.float32)

  b_in2 = b_in.reshape(1, H)
  b_ih2 = b_ih.reshape(1, 3 * H)
  b_hh2 = b_hh.reshape(1, 3 * H)
  b_out2 = b_out.reshape(1, H)

  h = pl.pallas_call(
      _in_body,
      grid=(NBLK,),
      in_specs=[_row_spec(H), _full_spec((H, H)), _full_spec((1, H))],
      out_specs=_row_spec(H),
      out_shape=jax.ShapeDtypeStruct((N, H), jnp.float32),
  )(xs, W_in, b_in2)

  for i in range(NLAYERS):
    m, gh = pl.pallas_call(
        _pre_body,
        grid=(NBLK,),
        in_specs=[_row_spec(H), _full_spec((H, H)), _full_spec((3 * H, H)),
                  _full_spec((1, 3 * H))],
        out_specs=[_row_spec(H), _row_spec(3 * H)],
        out_shape=[jax.ShapeDtypeStruct((N, H), jnp.float32),
                   jax.ShapeDtypeStruct((N, 3 * H), jnp.float32)],
    )(h, ggnn_w[i], w_hh, b_hh2)

    aggs = _make_sc_scatter()(m, src_p, dst_p, z128)

    h = pl.pallas_call(
        _post_body,
        grid=(NBLK,),
        in_specs=[_row_spec(H), _row_spec(H), _row_spec(H), _row_spec(3 * H),
                  _full_spec((3 * H, H)), _full_spec((1, 3 * H))],
        out_specs=_row_spec(H),
        out_shape=jax.ShapeDtypeStruct((N, H), jnp.float32),
    )(aggs[0], aggs[1], h, gh, w_ih, b_ih2)

  out = pl.pallas_call(
      _out_body,
      grid=(NBLK,),
      in_specs=[_row_spec(H), _full_spec((H, H)), _full_spec((1, H))],
      out_specs=_row_spec(H),
      out_shape=jax.ShapeDtypeStruct((N, H), jnp.float32),
  )(h, W_out, b_out2)

  return out.reshape(B, N, H)


# speedup vs baseline: 1.8981x; 1.0255x over previous
import functools

import jax
import jax.numpy as jnp
from jax import lax
from jax.experimental import pallas as pl
from jax.experimental.pallas import tpu as pltpu
from jax.experimental.pallas import tpu_sc as plsc

N = 10000
H = 128
NLAYERS = 3
BR = 400
NBLK = N // BR

NW = 32
CHUNK = 128
NCH = 80
EPT = CHUNK * NCH
EPAD = NW * EPT
R = 10240
RPT = R // 16


def _dot(a, w):
  return lax.dot_general(a, w, (((1,), (1,)), ((), ())),
                         preferred_element_type=jnp.float32)


def _dotn(a, w):
  return lax.dot_general(a, w, (((1,), (0,)), ((), ())),
                         preferred_element_type=jnp.float32)


def _pre(h, gw_ref, whh_ref, bhh_ref, m_ref, gh_ref):
  m_ref[...] = _dotn(h, gw_ref[...])
  gh_ref[...] = _dot(h, whh_ref[...]) + bhh_ref[...]


def _gru(a0_ref, a1_ref, h_ref, gh_ref, wih_ref, bih_ref):
  agg = a0_ref[...] + a1_ref[...]
  gi = _dot(agg, wih_ref[...]) + bih_ref[...]
  gh = gh_ref[...]
  h = h_ref[...]
  r = jax.nn.sigmoid(gi[:, :H] + gh[:, :H])
  z = jax.nn.sigmoid(gi[:, H:2 * H] + gh[:, H:2 * H])
  n = jnp.tanh(gi[:, 2 * H:] + r * gh[:, 2 * H:])
  return (1.0 - z) * n + z * h


def _inpre_body(x_ref, w_ref, b_ref, gw_ref, whh_ref, bhh_ref,
                h_ref, m_ref, gh_ref):
  h = jnp.maximum(_dot(x_ref[...], w_ref[...]) + b_ref[...], 0.0)
  h_ref[...] = h
  _pre(h, gw_ref, whh_ref, bhh_ref, m_ref, gh_ref)


def _postpre_body(a0_ref, a1_ref, h_ref, gh_ref, wih_ref, bih_ref,
                  gw_ref, whh_ref, bhh_ref, ho_ref, m_ref, gho_ref):
  hn = _gru(a0_ref, a1_ref, h_ref, gh_ref, wih_ref, bih_ref)
  ho_ref[...] = hn
  _pre(hn, gw_ref, whh_ref, bhh_ref, m_ref, gho_ref)


def _postout_body(a0_ref, a1_ref, h_ref, gh_ref, wih_ref, bih_ref,
                  w_ref, b_ref, o_ref):
  hn = jnp.maximum(_gru(a0_ref, a1_ref, h_ref, gh_ref, wih_ref, bih_ref), 0.0)
  o_ref[...] = jnp.maximum(_dot(hn, w_ref[...]) + b_ref[...], 0.0)


def _row_spec(cols):
  return pl.BlockSpec((BR, cols), lambda i: (i, 0))


def _full_spec(shape):
  return pl.BlockSpec(shape, lambda i: tuple(0 for _ in shape))


def _make_sc_scatter():
  mesh = plsc.VectorSubcoreMesh(core_axis_name="c", subcore_axis_name="s")

  @functools.partial(
      pl.kernel,
      out_type=jax.ShapeDtypeStruct((2, R, H), jnp.float32),
      mesh=mesh,
      scratch_types=[
          pltpu.VMEM((NCH, CHUNK), jnp.int32),
          pltpu.VMEM((NCH, CHUNK), jnp.int32),
          pltpu.VMEM((CHUNK, H), jnp.float32),
          pltpu.VMEM_SHARED((R, H), jnp.float32),
          pltpu.SemaphoreType.DMA,
      ],
  )
  def sc_scatter(m_hbm, src_hbm, dst_hbm, z_hbm, out_hbm,
                 src_v, dst_v, rows_v, agg_sh, sem):
    c = lax.axis_index("c")
    s = lax.axis_index("s")
    wid = s * 2 + c

    for k in range(RPT // CHUNK):
      pltpu.sync_copy(z_hbm, agg_sh.at[pl.ds(s * RPT + k * CHUNK, CHUNK)])
    plsc.subcore_barrier()

    pltpu.sync_copy(src_hbm.at[pl.ds(wid * NCH, NCH)], src_v)
    pltpu.sync_copy(dst_hbm.at[pl.ds(wid * NCH, NCH)], dst_v)

    def body(j, carry):
      pltpu.async_copy(m_hbm.at[src_v.at[j]], rows_v, sem).wait()
      pltpu.sync_copy(rows_v, agg_sh.at[dst_v.at[j]], add=True)
      return carry

    lax.fori_loop(0, NCH, body, 0)
    plsc.subcore_barrier()

    for k in range(RPT // CHUNK):
      r0 = s * RPT + k * CHUNK
      pltpu.sync_copy(agg_sh.at[pl.ds(r0, CHUNK)],
                      out_hbm.at[c, pl.ds(r0, CHUNK)])

  return sc_scatter


_make_sc_scatter = functools.cache(_make_sc_scatter)


def kernel(x, edge_index, W_in, b_in, ggnn_w, w_ih, w_hh, b_ih, b_hh,
           W_out, b_out):
  B = x.shape[0]
  xs = x.reshape(B * N, x.shape[2])
  src = edge_index[0]
  dst = edge_index[1]
  E = src.shape[0]
  pad = EPAD - E
  src_p = jnp.concatenate([src, jnp.zeros((pad,), jnp.int32)]).reshape(-1, CHUNK)
  dst_p = jnp.concatenate([dst, jnp.full((pad,), R - 1, jnp.int32)]).reshape(-1, CHUNK)
  z128 = jnp.zeros((CHUNK, H), jnp.float32)

  b_in2 = b_in.reshape(1, H)
  b_ih2 = b_ih.reshape(1, 3 * H)
  b_hh2 = b_hh.reshape(1, 3 * H)
  b_out2 = b_out.reshape(1, H)

  h, m, gh = pl.pallas_call(
      _inpre_body,
      grid=(NBLK,),
      in_specs=[_row_spec(H), _full_spec((H, H)), _full_spec((1, H)),
                _full_spec((H, H)), _full_spec((3 * H, H)),
                _full_spec((1, 3 * H))],
      out_specs=[_row_spec(H), _row_spec(H), _row_spec(3 * H)],
      out_shape=[jax.ShapeDtypeStruct((N, H), jnp.float32),
                 jax.ShapeDtypeStruct((N, H), jnp.float32),
                 jax.ShapeDtypeStruct((N, 3 * H), jnp.float32)],
  )(xs, W_in, b_in2, ggnn_w[0], w_hh, b_hh2)

  for i in range(NLAYERS - 1):
    aggs = _make_sc_scatter()(m, src_p, dst_p, z128)
    h, m, gh = pl.pallas_call(
        _postpre_body,
        grid=(NBLK,),
        in_specs=[_row_spec(H), _row_spec(H), _row_spec(H), _row_spec(3 * H),
                  _full_spec((3 * H, H)), _full_spec((1, 3 * H)),
                  _full_spec((H, H)), _full_spec((3 * H, H)),
                  _full_spec((1, 3 * H))],
        out_specs=[_row_spec(H), _row_spec(H), _row_spec(3 * H)],
        out_shape=[jax.ShapeDtypeStruct((N, H), jnp.float32),
                   jax.ShapeDtypeStruct((N, H), jnp.float32),
                   jax.ShapeDtypeStruct((N, 3 * H), jnp.float32)],
    )(aggs[0], aggs[1], h, gh, w_ih, b_ih2, ggnn_w[i + 1], w_hh, b_hh2)

  aggs = _make_sc_scatter()(m, src_p, dst_p, z128)
  out = pl.pallas_call(
      _postout_body,
      grid=(NBLK,),
      in_specs=[_row_spec(H), _row_spec(H), _row_spec(H), _row_spec(3 * H),
                _full_spec((3 * H, H)), _full_spec((1, 3 * H)),
                _full_spec((H, H)), _full_spec((1, H))],
      out_specs=_row_spec(H),
      out_shape=jax.ShapeDtypeStruct((N, H), jnp.float32),
  )(aggs[0], aggs[1], h, gh, w_ih, b_ih2, W_out, b_out2)

  return out.reshape(B, N, H)
